# Initial kernel scaffold; baseline (speedup 1.0000x reference)
import jax
import jax.numpy as jnp
from jax.experimental import pallas as pl


def kernel(x_mark, minute_embed, hour_embed):
    raise NotImplementedError("write your pallas kernel here")


# speedup vs baseline: 5.9840x; 5.9840x over previous
import functools

import jax
import jax.numpy as jnp
from jax import lax
from jax.experimental import pallas as pl
from jax.experimental.pallas import tpu as pltpu
from jax.experimental.pallas import tpu_sc as plsc

D = 128
NMIN = 60
NHOUR = 24
NC = 2
NS = 16
NW = NC * NS
L = 16
CHUNK = 128
NFEAT = 5
MIN_CH = 3
HOUR_CH = 2


def _combine_kernel(minute_ref, hour_ref, out_ref):
    m = minute_ref[...]
    h = hour_ref[...]
    c = m[:, None, :] + h[None, :, :]
    out_ref[...] = c


def _combined_table(minute_embed, hour_embed):
    c = pl.pallas_call(
        _combine_kernel,
        out_shape=jax.ShapeDtypeStruct((NMIN, NHOUR, D), jnp.float32),
    )(minute_embed, hour_embed)
    return c.reshape(NMIN * NHOUR, D)


def _make_gather(n_tok):
    assert n_tok % (NW * CHUNK) == 0
    tok_per_w = n_tok // NW
    n_chunks = tok_per_w // CHUNK
    mesh = plsc.VectorSubcoreMesh(
        core_axis_name="c", subcore_axis_name="s", num_cores=NC, num_subcores=NS
    )

    @functools.partial(
        pl.kernel,
        out_type=jax.ShapeDtypeStruct((n_tok, D), jnp.float32),
        mesh=mesh,
        scratch_types=[
            pltpu.VMEM((CHUNK * NFEAT,), jnp.float32),
            pltpu.VMEM((CHUNK,), jnp.int32),
            pltpu.VMEM((CHUNK, D), jnp.float32),
            pltpu.SemaphoreType.DMA,
        ],
        compiler_params=pltpu.CompilerParams(needs_layout_passes=False),
    )
    def gather(x_hbm, c_hbm, out_hbm, xbuf, idxbuf, rows, sem):
        wid = lax.axis_index("s") * NC + lax.axis_index("c")
        w_base = wid * tok_per_w

        def chunk_body(i, carry):
            base = w_base + i * CHUNK
            pltpu.sync_copy(x_hbm.at[pl.ds(base * NFEAT, CHUNK * NFEAT)], xbuf)
            for j in range(CHUNK // L):
                lanes = lax.iota(jnp.int32, L) * NFEAT + (L * NFEAT * j)
                fm = plsc.load_gather(xbuf, [lanes + MIN_CH])
                fh = plsc.load_gather(xbuf, [lanes + HOUR_CH])
                mi = (fm * 59.0).astype(jnp.int32)
                hi = (fh * 23.0).astype(jnp.int32)
                idxbuf[pl.ds(L * j, L)] = mi * NHOUR + hi
            pltpu.async_copy(c_hbm.at[idxbuf], rows, sem).wait()
            pltpu.sync_copy(rows, out_hbm.at[pl.ds(base, CHUNK)])
            return carry

        lax.fori_loop(0, n_chunks, chunk_body, 0)

    return gather


def kernel(x_mark, minute_embed, hour_embed):
    b, t, _ = x_mark.shape
    n_tok = b * t
    c_table = _combined_table(minute_embed, hour_embed)
    out = _make_gather(n_tok)(x_mark.reshape(-1), c_table)
    return out.reshape(b, t, D)


# speedup vs baseline: 7.3731x; 1.2321x over previous
import functools

import jax
import jax.numpy as jnp
from jax import lax
from jax.experimental import pallas as pl
from jax.experimental.pallas import tpu as pltpu
from jax.experimental.pallas import tpu_sc as plsc

D = 128
NMIN = 60
NHOUR = 24
NC = 2
NS = 16
NW = NC * NS
L = 16
CHUNK = 128
NFEAT = 5
MIN_CH = 3
HOUR_CH = 2


def _combine_kernel(minute_ref, hour_ref, out_ref):
    m = minute_ref[...]
    h = hour_ref[...]
    c = m[:, None, :] + h[None, :, :]
    out_ref[...] = c


def _combined_table(minute_embed, hour_embed):
    c = pl.pallas_call(
        _combine_kernel,
        out_shape=jax.ShapeDtypeStruct((NMIN, NHOUR, D), jnp.float32),
    )(minute_embed, hour_embed)
    return c.reshape(NMIN * NHOUR, D)


def _make_gather(n_tok):
    assert n_tok % (NW * CHUNK) == 0
    tok_per_w = n_tok // NW
    n_chunks = tok_per_w // CHUNK
    mesh = plsc.VectorSubcoreMesh(
        core_axis_name="c", subcore_axis_name="s", num_cores=NC, num_subcores=NS
    )

    n_groups = n_chunks // 2

    @functools.partial(
        pl.kernel,
        out_type=jax.ShapeDtypeStruct((n_tok, D), jnp.float32),
        mesh=mesh,
        scratch_types=[
            pltpu.VMEM((CHUNK * NFEAT,), jnp.float32),
            pltpu.VMEM((CHUNK * NFEAT,), jnp.float32),
            pltpu.VMEM((CHUNK,), jnp.int32),
            pltpu.VMEM((CHUNK,), jnp.int32),
            pltpu.VMEM((CHUNK, D), jnp.float32),
            pltpu.VMEM((CHUNK, D), jnp.float32),
            pltpu.SemaphoreType.DMA,
            pltpu.SemaphoreType.DMA,
            pltpu.SemaphoreType.DMA,
            pltpu.SemaphoreType.DMA,
        ],
        compiler_params=pltpu.CompilerParams(needs_layout_passes=False),
    )
    def gather(x_hbm, c_hbm, out_hbm, x0, x1, i0, i1, r0, r1, gs0, gs1, ss0, ss1):
        wid = lax.axis_index("s") * NC + lax.axis_index("c")
        w_base = wid * tok_per_w

        def prep(ci, xb, ib, rows, gsem):
            base = w_base + ci * CHUNK
            pltpu.sync_copy(x_hbm.at[pl.ds(base * NFEAT, CHUNK * NFEAT)], xb)
            for j in range(CHUNK // L):
                lanes = lax.iota(jnp.int32, L) * NFEAT + (L * NFEAT * j)
                fm = plsc.load_gather(xb, [lanes + MIN_CH])
                fh = plsc.load_gather(xb, [lanes + HOUR_CH])
                mi = (fm * 59.0).astype(jnp.int32)
                hi = (fh * 23.0).astype(jnp.int32)
                ib[pl.ds(L * j, L)] = mi * NHOUR + hi
            pltpu.async_copy(c_hbm.at[ib], rows, gsem)

        def wait_gather(ib, rows, gsem):
            pltpu.make_async_copy(c_hbm.at[ib], rows, gsem).wait()

        def scatter(ci, rows, ssem):
            base = w_base + ci * CHUNK
            pltpu.async_copy(rows, out_hbm.at[pl.ds(base, CHUNK)], ssem)

        def wait_scatter(ci, rows, ssem):
            base = w_base + ci * CHUNK
            pltpu.make_async_copy(rows, out_hbm.at[pl.ds(base, CHUNK)], ssem).wait()

        prep(0, x0, i0, r0, gs0)

        def body(g, carry):
            c0 = 2 * g

            @pl.when(g >= 1)
            def _():
                wait_scatter(c0 - 1, r1, ss1)

            prep(c0 + 1, x1, i1, r1, gs1)
            wait_gather(i0, r0, gs0)
            scatter(c0, r0, ss0)
            wait_gather(i1, r1, gs1)
            scatter(c0 + 1, r1, ss1)

            @pl.when(g < n_groups - 1)
            def _():
                wait_scatter(c0, r0, ss0)
                prep(c0 + 2, x0, i0, r0, gs0)

            return carry

        lax.fori_loop(0, n_groups, body, 0)
        wait_scatter(2 * n_groups - 2, r0, ss0)
        wait_scatter(2 * n_groups - 1, r1, ss1)

    return gather


def kernel(x_mark, minute_embed, hour_embed):
    b, t, _ = x_mark.shape
    n_tok = b * t
    c_table = _combined_table(minute_embed, hour_embed)
    out = _make_gather(n_tok)(x_mark.reshape(-1), c_table)
    return out.reshape(b, t, D)


# speedup vs baseline: 9.2538x; 1.2551x over previous
import functools

import jax
import jax.numpy as jnp
from jax import lax
from jax.experimental import pallas as pl
from jax.experimental.pallas import tpu as pltpu
from jax.experimental.pallas import tpu_sc as plsc

D = 128
NMIN = 60
NHOUR = 24
NC = 2
NS = 16
NW = NC * NS
L = 16
CHUNK = 128
NFEAT = 5
MIN_CH = 3
HOUR_CH = 2


def _combine_kernel(minute_ref, hour_ref, out_ref):
    m = minute_ref[...]
    h = hour_ref[...]
    c = m[:, None, :] + h[None, :, :]
    out_ref[...] = c


def _combined_table(minute_embed, hour_embed):
    c = pl.pallas_call(
        _combine_kernel,
        out_shape=jax.ShapeDtypeStruct((NMIN, NHOUR, D), jnp.float32),
    )(minute_embed, hour_embed)
    return c.reshape(NMIN * NHOUR, D)


def _make_gather(n_tok):
    assert n_tok % (NW * CHUNK) == 0
    tok_per_w = n_tok // NW
    n_chunks = tok_per_w // CHUNK
    mesh = plsc.VectorSubcoreMesh(
        core_axis_name="c", subcore_axis_name="s", num_cores=NC, num_subcores=NS
    )

    n_groups = n_chunks // 2

    @functools.partial(
        pl.kernel,
        out_type=jax.ShapeDtypeStruct((n_tok, D), jnp.float32),
        mesh=mesh,
        scratch_types=[
            pltpu.VMEM((CHUNK * NFEAT,), jnp.float32),
            pltpu.VMEM((CHUNK * NFEAT,), jnp.float32),
            pltpu.VMEM((CHUNK,), jnp.int32),
            pltpu.VMEM((CHUNK,), jnp.int32),
            pltpu.VMEM((CHUNK, D), jnp.float32),
            pltpu.VMEM((CHUNK, D), jnp.float32),
            pltpu.SemaphoreType.DMA,
            pltpu.SemaphoreType.DMA,
            pltpu.SemaphoreType.DMA,
            pltpu.SemaphoreType.DMA,
            pltpu.VMEM_SHARED((NMIN * NHOUR, D), jnp.float32),
        ],
        compiler_params=pltpu.CompilerParams(needs_layout_passes=False),
    )
    def gather(x_hbm, c_hbm, out_hbm, x0, x1, i0, i1, r0, r1, gs0, gs1, ss0, ss1,
               c_sp):
        wid = lax.axis_index("s") * NC + lax.axis_index("c")
        w_base = wid * tok_per_w

        @pl.when(lax.axis_index("s") == 0)
        def _():
            pltpu.sync_copy(c_hbm, c_sp)

        plsc.subcore_barrier()

        def prep(ci, xb, ib, rows, gsem):
            base = w_base + ci * CHUNK
            pltpu.sync_copy(x_hbm.at[pl.ds(base * NFEAT, CHUNK * NFEAT)], xb)
            for j in range(CHUNK // L):
                lanes = lax.iota(jnp.int32, L) * NFEAT + (L * NFEAT * j)
                fm = plsc.load_gather(xb, [lanes + MIN_CH])
                fh = plsc.load_gather(xb, [lanes + HOUR_CH])
                mi = (fm * 59.0).astype(jnp.int32)
                hi = (fh * 23.0).astype(jnp.int32)
                ib[pl.ds(L * j, L)] = mi * NHOUR + hi
            pltpu.async_copy(c_sp.at[ib], rows, gsem)

        def wait_gather(ib, rows, gsem):
            pltpu.make_async_copy(c_sp.at[ib], rows, gsem).wait()

        def scatter(ci, rows, ssem):
            base = w_base + ci * CHUNK
            pltpu.async_copy(rows, out_hbm.at[pl.ds(base, CHUNK)], ssem)

        def wait_scatter(ci, rows, ssem):
            base = w_base + ci * CHUNK
            pltpu.make_async_copy(rows, out_hbm.at[pl.ds(base, CHUNK)], ssem).wait()

        prep(0, x0, i0, r0, gs0)

        def body(g, carry):
            c0 = 2 * g

            @pl.when(g >= 1)
            def _():
                wait_scatter(c0 - 1, r1, ss1)

            prep(c0 + 1, x1, i1, r1, gs1)
            wait_gather(i0, r0, gs0)
            scatter(c0, r0, ss0)
            wait_gather(i1, r1, gs1)
            scatter(c0 + 1, r1, ss1)

            @pl.when(g < n_groups - 1)
            def _():
                wait_scatter(c0, r0, ss0)
                prep(c0 + 2, x0, i0, r0, gs0)

            return carry

        lax.fori_loop(0, n_groups, body, 0)
        wait_scatter(2 * n_groups - 2, r0, ss0)
        wait_scatter(2 * n_groups - 1, r1, ss1)

    return gather


def kernel(x_mark, minute_embed, hour_embed):
    b, t, _ = x_mark.shape
    n_tok = b * t
    c_table = _combined_table(minute_embed, hour_embed)
    out = _make_gather(n_tok)(x_mark.reshape(-1), c_table)
    return out.reshape(b, t, D)
